# baseline (speedup 1.0000x reference)
import functools

import jax
import jax.numpy as jnp
from jax import lax
from jax.experimental import pallas as pl
from jax.experimental.pallas import tpu as pltpu
from jax.experimental.pallas import tpu_sc as plsc

N_EMBD = 1024
HIDDEN = 4 * N_EMBD
NUM_EXPERTS = 8
TOP_K = 2
SEQ = 2048
NA = SEQ * TOP_K

BM = 256
NTILES = 24
NP = NTILES * BM
NTE_PAD = 32
BH = HIDDEN // 2
NH = 2

NC = 2
NS = 16
NW = NC * NS
CHUNK = NA // NW
TPW = SEQ // NW
NV = NA // 16


def _gelu_exact(x):
    return 0.5 * x * (1.0 + jax.lax.erf(x * 0.7071067811865476))


def _bc(s, dtype=jnp.int32):
    return jax.lax.broadcast_in_dim(jnp.asarray(s, dtype), (16,), ())


def _router_body(x_ref, wg_ref, logits_ref, eids_ref, wts_ref):
    x = x_ref[...]
    wg = wg_ref[...]
    logits = jax.lax.dot_general(
        x, wg, (((1,), (1,)), ((), ())),
        preferred_element_type=jnp.float32,
        precision=jax.lax.Precision.DEFAULT)
    logits_ref[...] = logits
    m = jnp.max(logits, axis=-1, keepdims=True)
    p = jnp.exp(logits - m)
    p = p / jnp.sum(p, axis=-1, keepdims=True)
    lanes = jax.lax.broadcasted_iota(jnp.int32, p.shape, 1)
    p1 = jnp.max(p, axis=-1, keepdims=True)
    i1 = jnp.min(jnp.where(p == p1, lanes, NUM_EXPERTS), axis=-1, keepdims=True)
    oh1 = lanes == i1
    pm = jnp.where(oh1, -jnp.inf, p)
    p2 = jnp.max(pm, axis=-1, keepdims=True)
    i2 = jnp.min(jnp.where(pm == p2, lanes, NUM_EXPERTS), axis=-1, keepdims=True)
    denom = p1 + p2
    k_lanes = jax.lax.broadcasted_iota(jnp.int32, (SEQ, TOP_K), 1)
    eids_ref[...] = jnp.where(k_lanes == 0, i1, i2)
    wts_ref[...] = jnp.where(k_lanes == 0, p1 / denom, p2 / denom)


def _router(x, Wg):
    return pl.pallas_call(
        _router_body,
        out_shape=(
            jax.ShapeDtypeStruct((SEQ, NUM_EXPERTS), jnp.float32),
            jax.ShapeDtypeStruct((SEQ, TOP_K), jnp.int32),
            jax.ShapeDtypeStruct((SEQ, TOP_K), jnp.float32),
        ),
    )(x, Wg)


def _dispatch_body(eids_hbm, x_hbm, pos_hbm, texp_hbm, rp_hbm, xs_hbm,
                   e_all, pos_v, peven, podd, texp_v, rp_v, xrows,
                   sem1, sem2):
    wid = lax.axis_index("s") * NC + lax.axis_index("c")
    pltpu.sync_copy(eids_hbm, e_all)

    iota = jax.lax.broadcasted_iota(jnp.int32, (16,), 0)
    zeros = jnp.zeros((16,), jnp.int32)
    myv0 = wid * (CHUNK // 16)

    def body(j, carry):
        accs = list(carry)
        v = e_all[pl.ds(j * 16, 16)]
        inpre = _bc((j < myv0).astype(jnp.int32))
        for ex in range(NUM_EXPERTS):
            m = (v == _bc(ex)).astype(jnp.int32)
            accs[ex] = accs[ex] + m
            accs[NUM_EXPERTS + ex] = accs[NUM_EXPERTS + ex] + m * inpre
        return tuple(accs)

    init = tuple(zeros for _ in range(2 * NUM_EXPERTS))
    accs = jax.lax.fori_loop(0, NV, body, init)
    totals = [jnp.sum(accs[ex]) for ex in range(NUM_EXPERTS)]
    prefix = [jnp.sum(accs[NUM_EXPERTS + ex]) for ex in range(NUM_EXPERTS)]

    offs = []
    acc = jnp.int32(0)
    for ex in range(NUM_EXPERTS):
        offs.append(acc)
        acc = acc + ((totals[ex] + (BM - 1)) // BM) * BM

    run = list(prefix)
    for j in range(CHUNK // 16):
        v = e_all[pl.ds((myv0 + j) * 16, 16)]
        posv = zeros
        ones = jnp.ones((16,), jnp.int32)
        for ex in range(NUM_EXPERTS):
            m = v == _bc(ex)
            mi = m.astype(jnp.int32)
            cs = jnp.cumsum(mi)
            posv = jnp.where(m, _bc(offs[ex] + run[ex]) + cs - ones, posv)
            run[ex] = run[ex] + jnp.sum(mi)
        pos_v[pl.ds(j * 16, 16)] = posv
    pltpu.sync_copy(pos_v, pos_hbm.at[pl.ds(wid * CHUNK, CHUNK)])

    @pl.when(wid == 0)
    def _texp():
        starts = [offs[ex] // BM for ex in range(NUM_EXPERTS)]
        ones = jnp.ones((16,), jnp.int32)
        emax = jnp.full((16,), NUM_EXPERTS - 1, jnp.int32)
        for c in range(NTE_PAD // 16):
            tvec = iota + jnp.full((16,), c * 16, jnp.int32)
            cnt = zeros
            runc = zeros
            for ex in range(NUM_EXPERTS):
                ge = (tvec >= _bc(starts[ex])).astype(jnp.int32)
                cnt = cnt + ge
                ne = (totals[ex] > 0).astype(jnp.int32)
                runc = runc + ge * _bc(ne)
            texp_v[pl.ds(c * 16, 16)] = jnp.minimum(
                jnp.maximum(cnt - ones, zeros), emax)
            rp_v[pl.ds(c * 16, 16)] = jnp.bitwise_and(runc, ones)
        pltpu.sync_copy(texp_v, texp_hbm)
        pltpu.sync_copy(rp_v, rp_hbm)

    two = jnp.full((16,), 2, jnp.int32)
    one = jnp.ones((16,), jnp.int32)
    for j in range(TPW // 16):
        idx = iota * two + jnp.full((16,), j * 32, jnp.int32)
        peven[pl.ds(j * 16, 16)] = plsc.load_gather(pos_v, [idx])
        podd[pl.ds(j * 16, 16)] = plsc.load_gather(pos_v, [idx + one])

    pltpu.sync_copy(x_hbm.at[pl.ds(wid * TPW, TPW)], xrows)
    c1 = pltpu.async_copy(xrows, xs_hbm.at[peven], sem1)
    c2 = pltpu.async_copy(xrows, xs_hbm.at[podd], sem2)
    c1.wait()
    c2.wait()


def _dispatch(eids_flat, x):
    mesh = plsc.VectorSubcoreMesh(core_axis_name="c", subcore_axis_name="s")
    f = functools.partial(
        pl.kernel, mesh=mesh,
        out_type=(
            jax.ShapeDtypeStruct((NA,), jnp.int32),
            jax.ShapeDtypeStruct((NTE_PAD,), jnp.int32),
            jax.ShapeDtypeStruct((NTE_PAD,), jnp.int32),
            jax.ShapeDtypeStruct((NP, N_EMBD), jnp.float32),
        ),
        scratch_types=[
            pltpu.VMEM((NA,), jnp.int32),
            pltpu.VMEM((CHUNK,), jnp.int32),
            pltpu.VMEM((TPW,), jnp.int32),
            pltpu.VMEM((TPW,), jnp.int32),
            pltpu.VMEM((NTE_PAD,), jnp.int32),
            pltpu.VMEM((NTE_PAD,), jnp.int32),
            pltpu.VMEM((TPW, N_EMBD), jnp.float32),
            pltpu.SemaphoreType.DMA,
            pltpu.SemaphoreType.DMA,
        ],
        compiler_params=pltpu.CompilerParams(needs_layout_passes=False),
    )(_dispatch_body)
    return f(eids_flat, x)


def _ffn_body(es_ref, rp_ref, xs_ref, w1_ref, w2_ref, ys_ref, w1b, w2b):
    t = pl.program_id(1)
    u = jnp.minimum(t, NTILES - 1)
    tile = jnp.maximum(t - 1, 0)
    eu = es_ref[u]
    et = es_ref[tile]

    @pl.when((t == 0) | (eu != et))
    def _convert():
        pu = rp_ref[u]
        w1b[pu] = w1_ref[0].astype(jnp.bfloat16)
        w2b[pu] = w2_ref[0].astype(jnp.bfloat16)

    @pl.when(t > 0)
    def _compute():
        pt = rp_ref[tile]
        x = xs_ref[...].astype(jnp.bfloat16)
        h = jnp.dot(x, w1b[pt], preferred_element_type=jnp.float32)
        h = _gelu_exact(h).astype(jnp.bfloat16)
        ys_ref[0] = jnp.dot(h, w2b[pt], preferred_element_type=jnp.float32)


def _ffn(tile_expert, run_par, Xs, W1, W2):
    grid_spec = pltpu.PrefetchScalarGridSpec(
        num_scalar_prefetch=2,
        grid=(NH, NTILES + 1),
        in_specs=[
            pl.BlockSpec((BM, N_EMBD),
                         lambda hb, t, es, rp: (jnp.maximum(t - 1, 0), 0)),
            pl.BlockSpec((1, N_EMBD, BH),
                         lambda hb, t, es, rp:
                         (es[jnp.minimum(t, NTILES - 1)], 0, hb)),
            pl.BlockSpec((1, BH, N_EMBD),
                         lambda hb, t, es, rp:
                         (es[jnp.minimum(t, NTILES - 1)], hb, 0)),
        ],
        out_specs=pl.BlockSpec((1, BM, N_EMBD),
                               lambda hb, t, es, rp:
                               (hb, jnp.maximum(t - 1, 0), 0)),
        scratch_shapes=[
            pltpu.VMEM((2, N_EMBD, BH), jnp.bfloat16),
            pltpu.VMEM((2, BH, N_EMBD), jnp.bfloat16),
        ],
    )
    return pl.pallas_call(
        _ffn_body,
        grid_spec=grid_spec,
        out_shape=jax.ShapeDtypeStruct((NH, NP, N_EMBD), jnp.float32),
        compiler_params=pltpu.CompilerParams(
            dimension_semantics=("arbitrary", "arbitrary")),
    )(tile_expert, run_par, Xs, W1, W2)


TCH = 16


def _combine_body(ys_hbm, pos_hbm, w_hbm, out_hbm,
                  pos_v, w_v, i00, i01, i10, i11, g00, g01, g10, g11, out_v,
                  s0, s1, s2, s3):
    wid = lax.axis_index("s") * NC + lax.axis_index("c")
    pltpu.sync_copy(pos_hbm.at[pl.ds(wid * CHUNK, CHUNK)], pos_v)
    pltpu.sync_copy(w_hbm.at[pl.ds(wid * CHUNK, CHUNK)], w_v)
    iota = jax.lax.broadcasted_iota(jnp.int32, (16,), 0)

    two = jnp.full((16,), 2, jnp.int32)
    one = jnp.ones((16,), jnp.int32)
    npv = jnp.full((16,), NP, jnp.int32)
    for ch in range(TPW // TCH):
        base = ch * 2 * TCH
        bvec = jnp.full((16,), base, jnp.int32)
        p0 = plsc.load_gather(pos_v, [iota * two + bvec])
        p1 = plsc.load_gather(pos_v, [iota * two + bvec + one])
        i00[...] = p0
        i01[...] = p0 + npv
        i10[...] = p1
        i11[...] = p1 + npv
        cps = [pltpu.async_copy(ys_hbm.at[i00], g00, s0),
               pltpu.async_copy(ys_hbm.at[i01], g01, s1),
               pltpu.async_copy(ys_hbm.at[i10], g10, s2),
               pltpu.async_copy(ys_hbm.at[i11], g11, s3)]
        for c in cps:
            c.wait()
        w0v = plsc.load_gather(w_v, [iota * two + bvec])
        w1v = plsc.load_gather(w_v, [iota * two + bvec + one])
        for r in range(TCH):
            w0 = _bc(w0v[r], jnp.float32)
            w1 = _bc(w1v[r], jnp.float32)

            def col(c2, _, r=r, w0=w0, w1=w1):
                sl = pl.ds(c2 * 16, 16)
                out_v[r, sl] = (w0 * (g00[r, sl] + g01[r, sl])
                                + w1 * (g10[r, sl] + g11[r, sl]))
                return 0

            jax.lax.fori_loop(0, N_EMBD // 16, col, 0)
        pltpu.sync_copy(out_v, out_hbm.at[pl.ds(wid * TPW + ch * TCH, TCH)])


def _combine(ys_flat, pos, w_flat):
    mesh = plsc.VectorSubcoreMesh(core_axis_name="c", subcore_axis_name="s")
    f = functools.partial(
        pl.kernel, mesh=mesh,
        out_type=jax.ShapeDtypeStruct((SEQ, N_EMBD), jnp.float32),
        scratch_types=[
            pltpu.VMEM((CHUNK,), jnp.int32),
            pltpu.VMEM((CHUNK,), jnp.float32),
            pltpu.VMEM((TCH,), jnp.int32),
            pltpu.VMEM((TCH,), jnp.int32),
            pltpu.VMEM((TCH,), jnp.int32),
            pltpu.VMEM((TCH,), jnp.int32),
            pltpu.VMEM((TCH, N_EMBD), jnp.float32),
            pltpu.VMEM((TCH, N_EMBD), jnp.float32),
            pltpu.VMEM((TCH, N_EMBD), jnp.float32),
            pltpu.VMEM((TCH, N_EMBD), jnp.float32),
            pltpu.VMEM((TCH, N_EMBD), jnp.float32),
            pltpu.SemaphoreType.DMA,
            pltpu.SemaphoreType.DMA,
            pltpu.SemaphoreType.DMA,
            pltpu.SemaphoreType.DMA,
        ],
        compiler_params=pltpu.CompilerParams(needs_layout_passes=False),
    )(_combine_body)
    return f(ys_flat, pos, w_flat)


def kernel(hidden_states, Wg, W1, W2):
    B, S, D = hidden_states.shape
    x = hidden_states.reshape(B * S, D)
    logits, eids, wts = _router(x, Wg)
    pos, texp, rpar, xs = _dispatch(eids.reshape(-1), x)
    ys = _ffn(texp, rpar, xs, W1, W2)
    out = _combine(ys.reshape(NH * NP, N_EMBD), pos, wts.reshape(-1))
    return out.reshape(B, S, D), logits


# speedup vs baseline: 1.1100x; 1.1100x over previous
import functools

import jax
import jax.numpy as jnp
from jax import lax
from jax.experimental import pallas as pl
from jax.experimental.pallas import tpu as pltpu
from jax.experimental.pallas import tpu_sc as plsc

N_EMBD = 1024
HIDDEN = 4 * N_EMBD
NUM_EXPERTS = 8
TOP_K = 2
SEQ = 2048
NA = SEQ * TOP_K

BM = 256
NTILES = 24
NP = NTILES * BM
NTE_PAD = 32
BH = HIDDEN // 2
NH = 2

NC = 2
NS = 16
NW = NC * NS
CHUNK = NA // NW
TPW = SEQ // NW
NV = NA // 16


def _gelu_exact(x):
    return 0.5 * x * (1.0 + jax.lax.erf(x * 0.7071067811865476))


def _bc(s, dtype=jnp.int32):
    return jax.lax.broadcast_in_dim(jnp.asarray(s, dtype), (16,), ())


def _router_body(x_ref, wg_ref, logits_ref, eids_ref, wts_ref):
    x = x_ref[...]
    wg = wg_ref[...]
    logits = jax.lax.dot_general(
        x, wg, (((1,), (1,)), ((), ())),
        preferred_element_type=jnp.float32,
        precision=jax.lax.Precision.DEFAULT)
    logits_ref[...] = logits
    m = jnp.max(logits, axis=-1, keepdims=True)
    p = jnp.exp(logits - m)
    p = p / jnp.sum(p, axis=-1, keepdims=True)
    lanes = jax.lax.broadcasted_iota(jnp.int32, p.shape, 1)
    p1 = jnp.max(p, axis=-1, keepdims=True)
    i1 = jnp.min(jnp.where(p == p1, lanes, NUM_EXPERTS), axis=-1, keepdims=True)
    oh1 = lanes == i1
    pm = jnp.where(oh1, -jnp.inf, p)
    p2 = jnp.max(pm, axis=-1, keepdims=True)
    i2 = jnp.min(jnp.where(pm == p2, lanes, NUM_EXPERTS), axis=-1, keepdims=True)
    denom = p1 + p2
    k_lanes = jax.lax.broadcasted_iota(jnp.int32, (SEQ, TOP_K), 1)
    eids_ref[...] = jnp.where(k_lanes == 0, i1, i2)
    wts_ref[...] = jnp.where(k_lanes == 0, p1 / denom, p2 / denom)


def _router(x, Wg):
    return pl.pallas_call(
        _router_body,
        out_shape=(
            jax.ShapeDtypeStruct((SEQ, NUM_EXPERTS), jnp.float32),
            jax.ShapeDtypeStruct((SEQ, TOP_K), jnp.int32),
            jax.ShapeDtypeStruct((SEQ, TOP_K), jnp.float32),
        ),
    )(x, Wg)


def _dispatch_body(eids_hbm, x_hbm, pos_hbm, texp_hbm, rp_hbm, xs_hbm,
                   e_all, pos_v, peven, podd, texp_v, rp_v, xrows,
                   sem1, sem2):
    wid = lax.axis_index("s") * NC + lax.axis_index("c")
    pltpu.sync_copy(eids_hbm, e_all)

    iota = jax.lax.broadcasted_iota(jnp.int32, (16,), 0)
    zeros = jnp.zeros((16,), jnp.int32)
    myv0 = wid * (CHUNK // 16)

    def body(j, carry):
        accs = list(carry)
        v = e_all[pl.ds(j * 16, 16)]
        inpre = _bc((j < myv0).astype(jnp.int32))
        for ex in range(NUM_EXPERTS):
            m = (v == _bc(ex)).astype(jnp.int32)
            accs[ex] = accs[ex] + m
            accs[NUM_EXPERTS + ex] = accs[NUM_EXPERTS + ex] + m * inpre
        return tuple(accs)

    init = tuple(zeros for _ in range(2 * NUM_EXPERTS))
    accs = jax.lax.fori_loop(0, NV, body, init)
    totals = [jnp.sum(accs[ex]) for ex in range(NUM_EXPERTS)]
    prefix = [jnp.sum(accs[NUM_EXPERTS + ex]) for ex in range(NUM_EXPERTS)]

    offs = []
    acc = jnp.int32(0)
    for ex in range(NUM_EXPERTS):
        offs.append(acc)
        acc = acc + ((totals[ex] + (BM - 1)) // BM) * BM

    run = list(prefix)
    for j in range(CHUNK // 16):
        v = e_all[pl.ds((myv0 + j) * 16, 16)]
        posv = zeros
        ones = jnp.ones((16,), jnp.int32)
        for ex in range(NUM_EXPERTS):
            m = v == _bc(ex)
            mi = m.astype(jnp.int32)
            cs = jnp.cumsum(mi)
            posv = jnp.where(m, _bc(offs[ex] + run[ex]) + cs - ones, posv)
            run[ex] = run[ex] + jnp.sum(mi)
        pos_v[pl.ds(j * 16, 16)] = posv
    pltpu.sync_copy(pos_v, pos_hbm.at[pl.ds(wid * CHUNK, CHUNK)])

    @pl.when(wid == 0)
    def _texp():
        starts = [offs[ex] // BM for ex in range(NUM_EXPERTS)]
        ones = jnp.ones((16,), jnp.int32)
        emax = jnp.full((16,), NUM_EXPERTS - 1, jnp.int32)
        for c in range(NTE_PAD // 16):
            tvec = iota + jnp.full((16,), c * 16, jnp.int32)
            cnt = zeros
            runc = zeros
            for ex in range(NUM_EXPERTS):
                ge = (tvec >= _bc(starts[ex])).astype(jnp.int32)
                cnt = cnt + ge
                ne = (totals[ex] > 0).astype(jnp.int32)
                runc = runc + ge * _bc(ne)
            texp_v[pl.ds(c * 16, 16)] = jnp.minimum(
                jnp.maximum(cnt - ones, zeros), emax)
            rp_v[pl.ds(c * 16, 16)] = jnp.bitwise_and(runc, ones)
        pltpu.sync_copy(texp_v, texp_hbm)
        pltpu.sync_copy(rp_v, rp_hbm)

    two = jnp.full((16,), 2, jnp.int32)
    one = jnp.ones((16,), jnp.int32)
    for j in range(TPW // 16):
        idx = iota * two + jnp.full((16,), j * 32, jnp.int32)
        peven[pl.ds(j * 16, 16)] = plsc.load_gather(pos_v, [idx])
        podd[pl.ds(j * 16, 16)] = plsc.load_gather(pos_v, [idx + one])

    pltpu.sync_copy(x_hbm.at[pl.ds(wid * TPW, TPW)], xrows)
    c1 = pltpu.async_copy(xrows, xs_hbm.at[peven], sem1)
    c2 = pltpu.async_copy(xrows, xs_hbm.at[podd], sem2)
    c1.wait()
    c2.wait()


def _dispatch(eids_flat, x):
    mesh = plsc.VectorSubcoreMesh(core_axis_name="c", subcore_axis_name="s")
    f = functools.partial(
        pl.kernel, mesh=mesh,
        out_type=(
            jax.ShapeDtypeStruct((NA,), jnp.int32),
            jax.ShapeDtypeStruct((NTE_PAD,), jnp.int32),
            jax.ShapeDtypeStruct((NTE_PAD,), jnp.int32),
            jax.ShapeDtypeStruct((NP, N_EMBD), jnp.float32),
        ),
        scratch_types=[
            pltpu.VMEM((NA,), jnp.int32),
            pltpu.VMEM((CHUNK,), jnp.int32),
            pltpu.VMEM((TPW,), jnp.int32),
            pltpu.VMEM((TPW,), jnp.int32),
            pltpu.VMEM((NTE_PAD,), jnp.int32),
            pltpu.VMEM((NTE_PAD,), jnp.int32),
            pltpu.VMEM((TPW, N_EMBD), jnp.float32),
            pltpu.SemaphoreType.DMA,
            pltpu.SemaphoreType.DMA,
        ],
        compiler_params=pltpu.CompilerParams(needs_layout_passes=False),
    )(_dispatch_body)
    return f(eids_flat, x)


def _ffn_body(es_ref, rp_ref, xs_ref, w1_ref, w2_ref, ys_ref,
              w1s, w2s, w1b, w2b, sem1, sem2):
    hb = pl.program_id(0)
    t = pl.program_id(1)
    u = jnp.minimum(t, NTILES - 1)
    tile = jnp.maximum(t - 1, 0)
    eu = es_ref[u]
    et = es_ref[tile]

    def w_copies(z):
        ez = es_ref[z]
        pz = rp_ref[z]
        c1 = pltpu.make_async_copy(
            w1_ref.at[ez, :, pl.ds(hb * BH, BH)], w1s.at[pz], sem1.at[pz])
        c2 = pltpu.make_async_copy(
            w2_ref.at[ez, pl.ds(hb * BH, BH), :], w2s.at[pz], sem2.at[pz])
        return c1, c2

    @pl.when(t == 0)
    def _prologue():
        c1, c2 = w_copies(0)
        c1.start()
        c2.start()

        @pl.when(es_ref[1] != es_ref[0])
        def _b1():
            d1, d2 = w_copies(1)
            d1.start()
            d2.start()

    @pl.when((t == 0) | (eu != et))
    def _convert():
        pu = rp_ref[u]
        c1, c2 = w_copies(u)
        c1.wait()
        c2.wait()
        w1b[pu] = w1s[pu].astype(jnp.bfloat16)
        w2b[pu] = w2s[pu].astype(jnp.bfloat16)

    z = jnp.minimum(t + 2, NTILES - 1)

    @pl.when((t + 2 <= NTILES - 1) & (es_ref[z] != es_ref[jnp.maximum(z - 1, 0)]))
    def _prefetch():
        c1, c2 = w_copies(z)
        c1.start()
        c2.start()

    @pl.when(t > 0)
    def _compute():
        pt = rp_ref[tile]
        x = xs_ref[...].astype(jnp.bfloat16)
        h = jnp.dot(x, w1b[pt], preferred_element_type=jnp.float32)
        h = _gelu_exact(h).astype(jnp.bfloat16)
        ys_ref[0] = jnp.dot(h, w2b[pt], preferred_element_type=jnp.float32)


def _ffn(tile_expert, run_par, Xs, W1, W2):
    grid_spec = pltpu.PrefetchScalarGridSpec(
        num_scalar_prefetch=2,
        grid=(NH, NTILES + 1),
        in_specs=[
            pl.BlockSpec((BM, N_EMBD),
                         lambda hb, t, es, rp: (jnp.maximum(t - 1, 0), 0)),
            pl.BlockSpec(memory_space=pltpu.MemorySpace.HBM),
            pl.BlockSpec(memory_space=pltpu.MemorySpace.HBM),
        ],
        out_specs=pl.BlockSpec((1, BM, N_EMBD),
                               lambda hb, t, es, rp:
                               (hb, jnp.maximum(t - 1, 0), 0)),
        scratch_shapes=[
            pltpu.VMEM((2, N_EMBD, BH), jnp.float32),
            pltpu.VMEM((2, BH, N_EMBD), jnp.float32),
            pltpu.VMEM((2, N_EMBD, BH), jnp.bfloat16),
            pltpu.VMEM((2, BH, N_EMBD), jnp.bfloat16),
            pltpu.SemaphoreType.DMA((2,)),
            pltpu.SemaphoreType.DMA((2,)),
        ],
    )
    return pl.pallas_call(
        _ffn_body,
        grid_spec=grid_spec,
        out_shape=jax.ShapeDtypeStruct((NH, NP, N_EMBD), jnp.float32),
        compiler_params=pltpu.CompilerParams(
            dimension_semantics=("arbitrary", "arbitrary")),
    )(tile_expert, run_par, Xs, W1, W2)


TCH = 16


def _combine_body(ys_hbm, pos_hbm, w_hbm, out_hbm,
                  pos_v, w_v, i00, i01, i10, i11, g00, g01, g10, g11, out_v,
                  s0, s1, s2, s3):
    wid = lax.axis_index("s") * NC + lax.axis_index("c")
    pltpu.sync_copy(pos_hbm.at[pl.ds(wid * CHUNK, CHUNK)], pos_v)
    pltpu.sync_copy(w_hbm.at[pl.ds(wid * CHUNK, CHUNK)], w_v)
    iota = jax.lax.broadcasted_iota(jnp.int32, (16,), 0)

    two = jnp.full((16,), 2, jnp.int32)
    one = jnp.ones((16,), jnp.int32)
    npv = jnp.full((16,), NP, jnp.int32)
    for ch in range(TPW // TCH):
        base = ch * 2 * TCH
        bvec = jnp.full((16,), base, jnp.int32)
        p0 = plsc.load_gather(pos_v, [iota * two + bvec])
        p1 = plsc.load_gather(pos_v, [iota * two + bvec + one])
        i00[...] = p0
        i01[...] = p0 + npv
        i10[...] = p1
        i11[...] = p1 + npv
        cps = [pltpu.async_copy(ys_hbm.at[i00], g00, s0),
               pltpu.async_copy(ys_hbm.at[i01], g01, s1),
               pltpu.async_copy(ys_hbm.at[i10], g10, s2),
               pltpu.async_copy(ys_hbm.at[i11], g11, s3)]
        for c in cps:
            c.wait()
        w0v = plsc.load_gather(w_v, [iota * two + bvec])
        w1v = plsc.load_gather(w_v, [iota * two + bvec + one])
        for r in range(TCH):
            w0 = _bc(w0v[r], jnp.float32)
            w1 = _bc(w1v[r], jnp.float32)

            def col(c2, _, r=r, w0=w0, w1=w1):
                sl = pl.ds(c2 * 16, 16)
                out_v[r, sl] = (w0 * (g00[r, sl] + g01[r, sl])
                                + w1 * (g10[r, sl] + g11[r, sl]))
                return 0

            jax.lax.fori_loop(0, N_EMBD // 16, col, 0)
        pltpu.sync_copy(out_v, out_hbm.at[pl.ds(wid * TPW + ch * TCH, TCH)])


def _combine(ys_flat, pos, w_flat):
    mesh = plsc.VectorSubcoreMesh(core_axis_name="c", subcore_axis_name="s")
    f = functools.partial(
        pl.kernel, mesh=mesh,
        out_type=jax.ShapeDtypeStruct((SEQ, N_EMBD), jnp.float32),
        scratch_types=[
            pltpu.VMEM((CHUNK,), jnp.int32),
            pltpu.VMEM((CHUNK,), jnp.float32),
            pltpu.VMEM((TCH,), jnp.int32),
            pltpu.VMEM((TCH,), jnp.int32),
            pltpu.VMEM((TCH,), jnp.int32),
            pltpu.VMEM((TCH,), jnp.int32),
            pltpu.VMEM((TCH, N_EMBD), jnp.float32),
            pltpu.VMEM((TCH, N_EMBD), jnp.float32),
            pltpu.VMEM((TCH, N_EMBD), jnp.float32),
            pltpu.VMEM((TCH, N_EMBD), jnp.float32),
            pltpu.VMEM((TCH, N_EMBD), jnp.float32),
            pltpu.SemaphoreType.DMA,
            pltpu.SemaphoreType.DMA,
            pltpu.SemaphoreType.DMA,
            pltpu.SemaphoreType.DMA,
        ],
        compiler_params=pltpu.CompilerParams(needs_layout_passes=False),
    )(_combine_body)
    return f(ys_flat, pos, w_flat)


def kernel(hidden_states, Wg, W1, W2):
    B, S, D = hidden_states.shape
    x = hidden_states.reshape(B * S, D)
    logits, eids, wts = _router(x, Wg)
    pos, texp, rpar, xs = _dispatch(eids.reshape(-1), x)
    ys = _ffn(texp, rpar, xs, W1, W2)
    out = _combine(ys.reshape(NH * NP, N_EMBD), pos, wts.reshape(-1))
    return out.reshape(B, S, D), logits
